# baseline (speedup 1.0000x reference)
import functools

import jax
import jax.numpy as jnp
from jax import lax
from jax.experimental import pallas as pl
from jax.experimental.pallas import tpu as pltpu
from jax.experimental.pallas import tpu_sc as plsc

_TOPK = 8
_NE = 64
_NT = 8192
_D = 4096
_NC = 2
_NS = 16
_NW = _NC * _NS
_L = 16
_TPW = _NT // _NW
_GROUPS = _TPW // _L


def _pack_body(w_ref, x_ref, ko_ref):
    logits = lax.dot_general(
        w_ref[:], x_ref[:], (((1,), (1,)), ((), ())),
        preferred_element_type=jnp.float32)
    u = lax.bitcast_convert_type(logits, jnp.int32)
    eidx = lax.broadcasted_iota(jnp.int32, u.shape, 0)
    low6 = (jnp.int32(63) - eidx) ^ (jnp.right_shift(u, 31) & jnp.int32(63))
    ko_ref[0] = lax.bitcast_convert_type(
        (u & jnp.int32(-64)) | low6, jnp.float32)


_MBT = 128
_SPB = _TPW // _MBT


def _compute_keys(x, W):
    return pl.pallas_call(
        _pack_body,
        grid=(_NT // _MBT,),
        in_specs=[
            pl.BlockSpec((_NE, _D), lambda i: (0, 0)),
            pl.BlockSpec((_MBT, _D), lambda i: (i, 0)),
        ],
        out_specs=pl.BlockSpec(
            (1, _NE, _MBT), lambda i: (i // _SPB, 0, i % _SPB)),
        out_shape=jax.ShapeDtypeStruct((_NW, _NE, _TPW), jnp.float32),
    )(W, x)


def _make_topk():
    mesh = plsc.VectorSubcoreMesh(core_axis_name="c", subcore_axis_name="s")

    @functools.partial(
        pl.kernel, mesh=mesh,
        out_type=jax.ShapeDtypeStruct((_NW, _TOPK, _TPW), jnp.float32),
        scratch_types=[
            pltpu.VMEM((_NE, _TPW), jnp.float32),
            pltpu.VMEM((_TOPK, _TPW), jnp.float32),
        ],
    )
    def topk_kernel(k_hbm, s_hbm, kv, sv):
        wid = lax.axis_index("s") * _NC + lax.axis_index("c")
        pltpu.sync_copy(k_hbm.at[wid], kv)

        def group_body(g, carry):
            base = g * _L
            b = [jnp.full((_L,), -jnp.inf, jnp.float32)] * _TOPK
            for e in range(_NE):
                key = kv[e, pl.ds(base, _L)]
                nb = [jnp.maximum(b[0], key)]
                for i in range(1, _TOPK):
                    nb.append(jnp.maximum(b[i], jnp.minimum(key, b[i - 1])))
                b = nb
            for k in range(_TOPK):
                sv[k, pl.ds(base, _L)] = b[k]
            return carry

        lax.fori_loop(0, _GROUPS, group_body, 0)
        pltpu.sync_copy(sv, s_hbm.at[wid])

    return topk_kernel


_topk = _make_topk()


def _unpack_body(s_ref, w_ref, i_ref):
    u = lax.bitcast_convert_type(s_ref[:], jnp.int32)
    sgn6 = jnp.right_shift(u, 31) & jnp.int32(63)
    low6 = u & jnp.int32(63)
    i_ref[:] = (jnp.int32(63) - low6) ^ sgn6
    va = lax.bitcast_convert_type(u & jnp.int32(-64), jnp.float32)
    ex = jnp.exp(va - va[:, 0:1, :])
    w_ref[:] = ex / jnp.sum(ex, axis=1, keepdims=True)


def _unpack_softmax(s):
    return pl.pallas_call(
        _unpack_body,
        grid=(1,),
        in_specs=[pl.BlockSpec((_NW, _TOPK, _TPW), lambda i: (0, 0, 0))],
        out_specs=[
            pl.BlockSpec((_NW, _TOPK, _TPW), lambda i: (0, 0, 0)),
            pl.BlockSpec((_NW, _TOPK, _TPW), lambda i: (0, 0, 0)),
        ],
        out_shape=[
            jax.ShapeDtypeStruct((_NW, _TOPK, _TPW), jnp.float32),
            jax.ShapeDtypeStruct((_NW, _TOPK, _TPW), jnp.int32),
        ],
    )(s)


def kernel(x, W):
    keys = _compute_keys(x, W)
    sel = _topk(keys)
    w_t, i_t = _unpack_softmax(sel)
    weights = w_t.transpose(0, 2, 1).reshape(_NT, _TOPK)
    indices = i_t.transpose(0, 2, 1).reshape(_NT, _TOPK)
    return (weights, indices)


# speedup vs baseline: 1.4213x; 1.4213x over previous
import functools

import jax
import jax.numpy as jnp
from jax import lax
from jax.experimental import pallas as pl
from jax.experimental.pallas import tpu as pltpu
from jax.experimental.pallas import tpu_sc as plsc

_TOPK = 8
_NE = 64
_NT = 8192
_D = 4096
_NC = 2
_NS = 16
_NW = _NC * _NS
_L = 16
_TPW = _NT // _NW
_GROUPS = _TPW // _L


def _pack_body(w_ref, x_ref, ko_ref):
    logits = lax.dot_general(
        w_ref[:], x_ref[:], (((1,), (1,)), ((), ())),
        preferred_element_type=jnp.float32)
    u = lax.bitcast_convert_type(logits, jnp.int32)
    eidx = lax.broadcasted_iota(jnp.int32, u.shape, 0)
    low6 = (jnp.int32(63) - eidx) ^ (jnp.right_shift(u, 31) & jnp.int32(63))
    packed = lax.bitcast_convert_type(
        (u & jnp.int32(-64)) | low6, jnp.float32)
    for s in range(_MBT // _TPW):
        ko_ref[s] = packed[:, s * _TPW:(s + 1) * _TPW]


_MBT = 512
_SPB = _MBT // _TPW


def _compute_keys(x, W):
    return pl.pallas_call(
        _pack_body,
        grid=(_NT // _MBT,),
        in_specs=[
            pl.BlockSpec((_NE, _D), lambda i: (0, 0)),
            pl.BlockSpec((_MBT, _D), lambda i: (i, 0)),
        ],
        out_specs=pl.BlockSpec(
            (_SPB, _NE, _TPW), lambda i: (i, 0, 0)),
        out_shape=jax.ShapeDtypeStruct((_NW, _NE, _TPW), jnp.float32),
    )(W, x)


def _make_topk():
    mesh = plsc.VectorSubcoreMesh(core_axis_name="c", subcore_axis_name="s")

    @functools.partial(
        pl.kernel, mesh=mesh,
        out_type=jax.ShapeDtypeStruct((_NW, _TOPK, _TPW), jnp.float32),
        scratch_types=[
            pltpu.VMEM((_NE, _TPW), jnp.float32),
            pltpu.VMEM((_TOPK, _TPW), jnp.float32),
        ],
    )
    def topk_kernel(k_hbm, s_hbm, kv, sv):
        wid = lax.axis_index("s") * _NC + lax.axis_index("c")
        pltpu.sync_copy(k_hbm.at[wid], kv)

        def group_body(g, carry):
            base = g * _L
            b = [jnp.full((_L,), -jnp.inf, jnp.float32)] * _TOPK
            for e in range(_NE):
                key = kv[e, pl.ds(base, _L)]
                nb = [jnp.maximum(b[0], key)]
                for i in range(1, _TOPK):
                    nb.append(jnp.maximum(b[i], jnp.minimum(key, b[i - 1])))
                b = nb
            for k in range(_TOPK):
                sv[k, pl.ds(base, _L)] = b[k]
            return carry

        lax.fori_loop(0, _GROUPS, group_body, 0)
        pltpu.sync_copy(sv, s_hbm.at[wid])

    return topk_kernel


_topk = _make_topk()


def _unpack_body(s_ref, w_ref, i_ref):
    u = lax.bitcast_convert_type(s_ref[:], jnp.int32)
    sgn6 = jnp.right_shift(u, 31) & jnp.int32(63)
    low6 = u & jnp.int32(63)
    i_ref[:] = (jnp.int32(63) - low6) ^ sgn6
    va = lax.bitcast_convert_type(u & jnp.int32(-64), jnp.float32)
    ex = jnp.exp(va - va[:, 0:1, :])
    w_ref[:] = ex / jnp.sum(ex, axis=1, keepdims=True)


def _unpack_softmax(s):
    return pl.pallas_call(
        _unpack_body,
        grid=(1,),
        in_specs=[pl.BlockSpec((_NW, _TOPK, _TPW), lambda i: (0, 0, 0))],
        out_specs=[
            pl.BlockSpec((_NW, _TOPK, _TPW), lambda i: (0, 0, 0)),
            pl.BlockSpec((_NW, _TOPK, _TPW), lambda i: (0, 0, 0)),
        ],
        out_shape=[
            jax.ShapeDtypeStruct((_NW, _TOPK, _TPW), jnp.float32),
            jax.ShapeDtypeStruct((_NW, _TOPK, _TPW), jnp.int32),
        ],
    )(s)


def kernel(x, W):
    keys = _compute_keys(x, W)
    sel = _topk(keys)
    w_t, i_t = _unpack_softmax(sel)
    weights = w_t.transpose(0, 2, 1).reshape(_NT, _TOPK)
    indices = i_t.transpose(0, 2, 1).reshape(_NT, _TOPK)
    return (weights, indices)
